# baseline (speedup 1.0000x reference)
import functools
import jax
import jax.numpy as jnp
from jax import lax
from jax.experimental import pallas as pl
from jax.experimental.pallas import tpu as pltpu
from jax.experimental.pallas import tpu_sc as plsc

GRID_SIZE = 2
AREA_SPAN = 1.6

N = 4096
H = 256
NC = 2
NS = 16
NW = NC * NS
NPASS = 2
IPP = N // (NW * NPASS)
CAP = 128
FLUSH_AT = CAP - 16
DUMP = IPP * 4
ACCROWS = IPP * 4 + 16


def _extract_i32(ref, idx):
    v = ref[pl.ds((idx // 16) * 16, 16)]
    lane = lax.iota(jnp.int32, 16)
    return jnp.max(jnp.where(lane == idx % 16, v, 0))


def _extract_f32(ref, idx):
    v = ref[pl.ds((idx // 16) * 16, 16)]
    lane = lax.iota(jnp.int32, 16)
    return jnp.max(jnp.where(lane == idx % 16, v, jnp.float32(-3.0e38)))


def _sc_body(ht_hbm, px_hbm, py_hbm, jlo_hbm, jhi_hbm, out_hbm,
             px_v, py_v, jlo_v, jhi_v, slist, dlist, stage, acc, sem):
    c = lax.axis_index("c")
    s = lax.axis_index("s")
    w = s * NC + c

    pltpu.sync_copy(px_hbm, px_v)
    pltpu.sync_copy(py_hbm, py_v)
    pltpu.sync_copy(jlo_hbm, jlo_v)
    pltpu.sync_copy(jhi_hbm, jhi_v)

    lane = lax.iota(jnp.int32, 16)

    def refill():
        zeros16 = jnp.zeros((16,), jnp.int32)
        dump16 = jnp.full((16,), DUMP, jnp.int32)
        for k in range(CAP // 16):
            slist[pl.ds(k * 16, 16)] = zeros16
            dlist[pl.ds(k * 16, 16)] = dump16

    def flush():
        pltpu.async_copy(ht_hbm.at[slist], stage, sem).wait()

        def acc_group(g, carry):
            dvec = dlist[pl.ds(g * 16, 16)]
            for l in range(16):
                d = jnp.max(jnp.where(lane == l, dvec, 0))
                r = g * 16 + l
                for k in range(H // 16):
                    acc[d, pl.ds(k * 16, 16)] += stage[r, pl.ds(k * 16, 16)]
            return carry

        lax.fori_loop(0, CAP // 16, acc_group, jnp.int32(0))
        refill()

    refill()

    half = AREA_SPAN / 2.0
    cell = AREA_SPAN / GRID_SIZE
    eps = 0.01

    for p in range(NPASS):
        my_i0 = w * (IPP * NPASS) + p * IPP

        zrow = jnp.zeros((16,), jnp.float32)

        def zero_row(r, carry):
            for k in range(H // 16):
                acc[r, pl.ds(k * 16, 16)] = zrow
            return carry

        lax.fori_loop(0, ACCROWS, zero_row, jnp.int32(0))

        def i_body(i, wp):
            jlo = _extract_i32(jlo_v, i)
            jhi = _extract_i32(jhi_v, i)
            pxi = _extract_f32(px_v, i)
            pyi = _extract_f32(py_v, i)
            dbase = (i - my_i0) * 4

            def j_body(g, wp):
                jg = g * 16 + lane
                pxj = px_v[pl.ds(g * 16, 16)]
                pyj = py_v[pl.ds(g * 16, 16)]
                relx = pxj - pxi
                rely = pyj - pyi
                box = ((relx < half - eps) & (relx > -(half - eps))
                       & (rely < half - eps) & (rely > -(half - eps)))
                inrange = (jg >= jlo) & (jg < jhi) & (jg != i)
                within = box & inrange
                gid = (jnp.where(relx + half >= cell, 2, 0)
                       + jnp.where(rely + half >= cell, 1, 0))
                dest = dbase + gid
                plsc.store_compressed(slist.at[pl.ds(wp, 16)], jg,
                                      mask=within)
                plsc.store_compressed(dlist.at[pl.ds(wp, 16)], dest,
                                      mask=within)
                wp = wp + jnp.sum(within.astype(jnp.int32))

                def do_flush(_):
                    flush()
                    return jnp.int32(0)

                return lax.cond(wp >= FLUSH_AT, do_flush, lambda x: x, wp)

            glo = jlo // 16
            ghi = (jhi + 15) // 16
            return lax.fori_loop(glo, ghi, j_body, wp)

        lax.fori_loop(my_i0, my_i0 + IPP, i_body, jnp.int32(0))
        flush()

        pltpu.sync_copy(acc.at[pl.ds(0, IPP * 4)],
                        out_hbm.at[pl.ds(my_i0 * 4, IPP * 4)])


def kernel(ht, pos_t, same_scene_mask):
    ht2 = ht.reshape(N, H).astype(jnp.float32)
    pos = pos_t.reshape(N, 2).astype(jnp.float32)
    ssm = same_scene_mask.reshape(N).astype(jnp.int32)
    px = pos[:, 0]
    py = pos[:, 1]
    jlo = jnp.searchsorted(ssm, ssm, side="left").astype(jnp.int32)
    jhi = jnp.searchsorted(ssm, ssm, side="right").astype(jnp.int32)

    mesh = plsc.VectorSubcoreMesh(core_axis_name="c", subcore_axis_name="s")
    fn = functools.partial(
        pl.kernel,
        mesh=mesh,
        compiler_params=pltpu.CompilerParams(needs_layout_passes=False),
        out_type=jax.ShapeDtypeStruct((N * 4, H), jnp.float32),
        scratch_types=[
            pltpu.VMEM((N,), jnp.float32),
            pltpu.VMEM((N,), jnp.float32),
            pltpu.VMEM((N,), jnp.int32),
            pltpu.VMEM((N,), jnp.int32),
            pltpu.VMEM((CAP,), jnp.int32),
            pltpu.VMEM((CAP,), jnp.int32),
            pltpu.VMEM((CAP, H), jnp.float32),
            pltpu.VMEM((ACCROWS, H), jnp.float32),
            pltpu.SemaphoreType.DMA,
        ],
    )(_sc_body)
    out = fn(ht2, px, py, jlo, jhi)
    return out.reshape(N, 4, H)


# speedup vs baseline: 1.0099x; 1.0099x over previous
import functools
import jax
import jax.numpy as jnp
from jax import lax
from jax.experimental import pallas as pl
from jax.experimental.pallas import tpu as pltpu
from jax.experimental.pallas import tpu_sc as plsc

GRID_SIZE = 2
AREA_SPAN = 1.6

N = 4096
H = 256
NC = 2
NS = 16
NW = NC * NS
NPASS = 2
IPP = N // (NW * NPASS)
CAP = 128
FLUSH_AT = CAP - 16
DUMP = IPP * 4
ACCROWS = IPP * 4 + 16


def _extract_i32(ref, idx):
    v = ref[pl.ds((idx // 16) * 16, 16)]
    lane = lax.iota(jnp.int32, 16)
    return jnp.max(jnp.where(lane == idx % 16, v, 0))


def _extract_f32(ref, idx):
    v = ref[pl.ds((idx // 16) * 16, 16)]
    lane = lax.iota(jnp.int32, 16)
    return jnp.max(jnp.where(lane == idx % 16, v, jnp.float32(-3.0e38)))


def _sc_body(ht_hbm, px_hbm, py_hbm, jlo_hbm, jhi_hbm, out_hbm,
             px_v, py_v, jlo_v, jhi_v, slist, dlist, stage, acc, sem):
    c = lax.axis_index("c")
    s = lax.axis_index("s")
    w = s * NC + c

    pltpu.sync_copy(px_hbm, px_v)
    pltpu.sync_copy(py_hbm, py_v)
    pltpu.sync_copy(jlo_hbm, jlo_v)
    pltpu.sync_copy(jhi_hbm, jhi_v)

    lane = lax.iota(jnp.int32, 16)

    def refill():
        zeros16 = jnp.zeros((16,), jnp.int32)
        dump16 = jnp.full((16,), DUMP, jnp.int32)
        for k in range(CAP // 16):
            slist[pl.ds(k * 16, 16)] = zeros16
            dlist[pl.ds(k * 16, 16)] = dump16

    def flush():
        pltpu.async_copy(ht_hbm.at[slist], stage, sem).wait()

        def acc_group(g, carry):
            dvec = dlist[pl.ds(g * 16, 16)]
            for l in range(16):
                d = jnp.max(jnp.where(lane == l, dvec, 0))
                r = g * 16 + l
                for k in range(H // 16):
                    plsc.addupdate(acc.at[d, pl.ds(k * 16, 16)],
                                   stage[r, pl.ds(k * 16, 16)])
            return carry

        lax.fori_loop(0, CAP // 16, acc_group, jnp.int32(0))
        refill()

    refill()

    half = AREA_SPAN / 2.0
    cell = AREA_SPAN / GRID_SIZE
    eps = 0.01

    for p in range(NPASS):
        my_i0 = w * (IPP * NPASS) + p * IPP

        zrow = jnp.zeros((16,), jnp.float32)

        def zero_row(r, carry):
            for k in range(H // 16):
                acc[r, pl.ds(k * 16, 16)] = zrow
            return carry

        lax.fori_loop(0, ACCROWS, zero_row, jnp.int32(0))

        def i_body(i, wp):
            jlo = _extract_i32(jlo_v, i)
            jhi = _extract_i32(jhi_v, i)
            pxi = _extract_f32(px_v, i)
            pyi = _extract_f32(py_v, i)
            dbase = (i - my_i0) * 4

            def j_body(g, wp):
                jg = g * 16 + lane
                pxj = px_v[pl.ds(g * 16, 16)]
                pyj = py_v[pl.ds(g * 16, 16)]
                relx = pxj - pxi
                rely = pyj - pyi
                box = ((relx < half - eps) & (relx > -(half - eps))
                       & (rely < half - eps) & (rely > -(half - eps)))
                inrange = (jg >= jlo) & (jg < jhi) & (jg != i)
                within = box & inrange
                gid = (jnp.where(relx + half >= cell, 2, 0)
                       + jnp.where(rely + half >= cell, 1, 0))
                dest = dbase + gid
                plsc.store_compressed(slist.at[pl.ds(wp, 16)], jg,
                                      mask=within)
                plsc.store_compressed(dlist.at[pl.ds(wp, 16)], dest,
                                      mask=within)
                wp = wp + jnp.sum(within.astype(jnp.int32))

                def do_flush(_):
                    flush()
                    return jnp.int32(0)

                return lax.cond(wp >= FLUSH_AT, do_flush, lambda x: x, wp)

            glo = jlo // 16
            ghi = (jhi + 15) // 16
            return lax.fori_loop(glo, ghi, j_body, wp)

        lax.fori_loop(my_i0, my_i0 + IPP, i_body, jnp.int32(0))
        flush()

        pltpu.sync_copy(acc.at[pl.ds(0, IPP * 4)],
                        out_hbm.at[pl.ds(my_i0 * 4, IPP * 4)])


def kernel(ht, pos_t, same_scene_mask):
    ht2 = ht.reshape(N, H).astype(jnp.float32)
    pos = pos_t.reshape(N, 2).astype(jnp.float32)
    ssm = same_scene_mask.reshape(N).astype(jnp.int32)
    px = pos[:, 0]
    py = pos[:, 1]
    jlo = jnp.searchsorted(ssm, ssm, side="left").astype(jnp.int32)
    jhi = jnp.searchsorted(ssm, ssm, side="right").astype(jnp.int32)

    mesh = plsc.VectorSubcoreMesh(core_axis_name="c", subcore_axis_name="s")
    fn = functools.partial(
        pl.kernel,
        mesh=mesh,
        compiler_params=pltpu.CompilerParams(needs_layout_passes=False),
        out_type=jax.ShapeDtypeStruct((N * 4, H), jnp.float32),
        scratch_types=[
            pltpu.VMEM((N,), jnp.float32),
            pltpu.VMEM((N,), jnp.float32),
            pltpu.VMEM((N,), jnp.int32),
            pltpu.VMEM((N,), jnp.int32),
            pltpu.VMEM((CAP,), jnp.int32),
            pltpu.VMEM((CAP,), jnp.int32),
            pltpu.VMEM((CAP, H), jnp.float32),
            pltpu.VMEM((ACCROWS, H), jnp.float32),
            pltpu.SemaphoreType.DMA,
        ],
    )(_sc_body)
    out = fn(ht2, px, py, jlo, jhi)
    return out.reshape(N, 4, H)


# speedup vs baseline: 1.0115x; 1.0016x over previous
import functools
import jax
import jax.numpy as jnp
from jax import lax
from jax.experimental import pallas as pl
from jax.experimental.pallas import tpu as pltpu
from jax.experimental.pallas import tpu_sc as plsc

GRID_SIZE = 2
AREA_SPAN = 1.6

N = 4096
H = 256
NC = 2
NS = 16
NW = NC * NS
NPASS = 2
IPP = N // (NW * NPASS)
CAP = 128
FLUSH_AT = CAP - 16
DUMP = IPP * 4
ACCROWS = IPP * 4 + 16


def _sc_body(ht_hbm, px_hbm, py_hbm, jlo_hbm, jhi_hbm, out_hbm,
             px_v, py_v, jlo_v, jhi_v, slist, dlist, stage, acc, sem):
    c = lax.axis_index("c")
    s = lax.axis_index("s")
    w = s * NC + c

    pltpu.sync_copy(px_hbm, px_v.at[pl.ds(0, N)])
    pltpu.sync_copy(py_hbm, py_v.at[pl.ds(0, N)])
    pltpu.sync_copy(jlo_hbm, jlo_v.at[pl.ds(0, N)])
    pltpu.sync_copy(jhi_hbm, jhi_v.at[pl.ds(0, N)])

    lane = lax.iota(jnp.int32, 16)

    def refill():
        zeros16 = jnp.zeros((16,), jnp.int32)
        dump16 = jnp.full((16,), DUMP, jnp.int32)
        for k in range(CAP // 16):
            slist[pl.ds(k * 16, 16)] = zeros16
            dlist[pl.ds(k * 16, 16)] = dump16

    def flush():
        pltpu.async_copy(ht_hbm.at[slist], stage, sem).wait()

        def acc_group(g, carry):
            for l in range(16):
                r = g * 16 + l
                d = dlist[pl.ds(r, 16)][0]
                for k in range(H // 16):
                    plsc.addupdate(acc.at[d, pl.ds(k * 16, 16)],
                                   stage[r, pl.ds(k * 16, 16)])
            return carry

        lax.fori_loop(0, CAP // 16, acc_group, jnp.int32(0))
        refill()

    refill()

    half = AREA_SPAN / 2.0
    cell = AREA_SPAN / GRID_SIZE
    eps = 0.01

    for p in range(NPASS):
        my_i0 = w * (IPP * NPASS) + p * IPP

        zrow = jnp.zeros((16,), jnp.float32)

        def zero_row(r, carry):
            for k in range(H // 16):
                acc[r, pl.ds(k * 16, 16)] = zrow
            return carry

        lax.fori_loop(0, ACCROWS, zero_row, jnp.int32(0))

        def i_body(i, wp):
            jlo = jlo_v[pl.ds(i, 16)][0]
            jhi = jhi_v[pl.ds(i, 16)][0]
            pxi = px_v[pl.ds(i, 16)][0]
            pyi = py_v[pl.ds(i, 16)][0]
            dbase = (i - my_i0) * 4

            def j_body(g, wp):
                jg = g * 16 + lane
                pxj = px_v[pl.ds(g * 16, 16)]
                pyj = py_v[pl.ds(g * 16, 16)]
                relx = pxj - pxi
                rely = pyj - pyi
                box = ((relx < half - eps) & (relx > -(half - eps))
                       & (rely < half - eps) & (rely > -(half - eps)))
                inrange = (jg >= jlo) & (jg < jhi) & (jg != i)
                within = box & inrange
                gid = (jnp.where(relx + half >= cell, 2, 0)
                       + jnp.where(rely + half >= cell, 1, 0))
                dest = dbase + gid
                plsc.store_compressed(slist.at[pl.ds(wp, 16)], jg,
                                      mask=within)
                plsc.store_compressed(dlist.at[pl.ds(wp, 16)], dest,
                                      mask=within)
                wp = wp + jnp.sum(within.astype(jnp.int32))

                def do_flush(_):
                    flush()
                    return jnp.int32(0)

                return lax.cond(wp >= FLUSH_AT, do_flush, lambda x: x, wp)

            glo = jlo // 16
            ghi = (jhi + 15) // 16
            return lax.fori_loop(glo, ghi, j_body, wp)

        lax.fori_loop(my_i0, my_i0 + IPP, i_body, jnp.int32(0))
        flush()

        pltpu.sync_copy(acc.at[pl.ds(0, IPP * 4)],
                        out_hbm.at[pl.ds(my_i0 * 4, IPP * 4)])


def kernel(ht, pos_t, same_scene_mask):
    ht2 = ht.reshape(N, H).astype(jnp.float32)
    pos = pos_t.reshape(N, 2).astype(jnp.float32)
    ssm = same_scene_mask.reshape(N).astype(jnp.int32)
    px = pos[:, 0]
    py = pos[:, 1]
    jlo = jnp.searchsorted(ssm, ssm, side="left").astype(jnp.int32)
    jhi = jnp.searchsorted(ssm, ssm, side="right").astype(jnp.int32)

    mesh = plsc.VectorSubcoreMesh(core_axis_name="c", subcore_axis_name="s")
    fn = functools.partial(
        pl.kernel,
        mesh=mesh,
        compiler_params=pltpu.CompilerParams(needs_layout_passes=False),
        out_type=jax.ShapeDtypeStruct((N * 4, H), jnp.float32),
        scratch_types=[
            pltpu.VMEM((N + 16,), jnp.float32),
            pltpu.VMEM((N + 16,), jnp.float32),
            pltpu.VMEM((N + 16,), jnp.int32),
            pltpu.VMEM((N + 16,), jnp.int32),
            pltpu.VMEM((CAP,), jnp.int32),
            pltpu.VMEM((CAP + 16,), jnp.int32),
            pltpu.VMEM((CAP, H), jnp.float32),
            pltpu.VMEM((ACCROWS, H), jnp.float32),
            pltpu.SemaphoreType.DMA,
        ],
    )(_sc_body)
    out = fn(ht2, px, py, jlo, jhi)
    return out.reshape(N, 4, H)
